# Initial kernel scaffold; baseline (speedup 1.0000x reference)
import jax
import jax.numpy as jnp
from jax.experimental import pallas as pl


def kernel(input_ids, attention_mask, W):
    raise NotImplementedError("write your pallas kernel here")


# speedup vs baseline: 6.3652x; 6.3652x over previous
import jax
import jax.numpy as jnp
from jax.experimental import pallas as pl
from jax.experimental.pallas import tpu as pltpu
from jax.experimental.pallas import tpu_sc as plsc

BATCH = 1024
SEQ = 200
DIM = 128

NUM_IDS = BATCH * SEQ
WINDOW = 128
assert NUM_IDS % WINDOW == 0


def _sc_gather(W, flat_ids):
    mesh = plsc.VectorSubcoreMesh(core_axis_name="c", subcore_axis_name="s")

    @pl.kernel(
        out_type=jax.ShapeDtypeStruct((NUM_IDS, DIM), W.dtype),
        mesh=mesh,
    )
    def gather_kernel(w_hbm, ids_hbm, out_hbm):
        def body(ids_vmem, out_vmem):
            pltpu.sync_copy(w_hbm.at[ids_vmem.at[0]], out_vmem)

        pltpu.emit_pipeline(
            body,
            grid=(NUM_IDS // WINDOW,),
            in_specs=[pl.BlockSpec((1, WINDOW), index_map=lambda i: (0, i))],
            out_specs=[pl.BlockSpec((WINDOW, DIM), index_map=lambda i: (i, 0))],
            core_axis_name=("c", "s"),
            dimension_semantics=(pltpu.PARALLEL,),
        )(ids_hbm, out_hbm)

    return gather_kernel(W, flat_ids)


def kernel(input_ids, attention_mask, W):
    flat_ids = input_ids.reshape(1, NUM_IDS)
    out = _sc_gather(W, flat_ids)
    token_embeddings = out.reshape(BATCH, SEQ, DIM)
    return (input_ids, token_embeddings, attention_mask)


# speedup vs baseline: 7.4919x; 1.1770x over previous
import jax
import jax.numpy as jnp
from jax.experimental import pallas as pl
from jax.experimental.pallas import tpu as pltpu
from jax.experimental.pallas import tpu_sc as plsc

BATCH = 1024
SEQ = 200
DIM = 128

NUM_IDS = BATCH * SEQ
WINDOW = 256
assert NUM_IDS % WINDOW == 0


def _sc_gather(W, flat_ids):
    mesh = plsc.VectorSubcoreMesh(core_axis_name="c", subcore_axis_name="s")

    @pl.kernel(
        out_type=jax.ShapeDtypeStruct((NUM_IDS, DIM), W.dtype),
        mesh=mesh,
    )
    def gather_kernel(w_hbm, ids_hbm, out_hbm):
        def body(ids_vmem, out_vmem):
            pltpu.sync_copy(w_hbm.at[ids_vmem.at[0]], out_vmem)

        pltpu.emit_pipeline(
            body,
            grid=(NUM_IDS // WINDOW,),
            in_specs=[pl.BlockSpec((1, WINDOW), index_map=lambda i: (0, i))],
            out_specs=[pl.BlockSpec((WINDOW, DIM), index_map=lambda i: (i, 0))],
            core_axis_name=("c", "s"),
            dimension_semantics=(pltpu.PARALLEL,),
        )(ids_hbm, out_hbm)

    return gather_kernel(W, flat_ids)


def kernel(input_ids, attention_mask, W):
    flat_ids = input_ids.reshape(1, NUM_IDS)
    out = _sc_gather(W, flat_ids)
    token_embeddings = out.reshape(BATCH, SEQ, DIM)
    return (input_ids, token_embeddings, attention_mask)


# speedup vs baseline: 7.7601x; 1.0358x over previous
import functools

import jax
import jax.numpy as jnp
from jax import lax
from jax.experimental import pallas as pl
from jax.experimental.pallas import tpu as pltpu
from jax.experimental.pallas import tpu_sc as plsc

BATCH = 1024
SEQ = 200
DIM = 128

NUM_IDS = BATCH * SEQ
NC, NS = 2, 16
NW = NC * NS
IDS_PER_W = NUM_IDS // NW
CHUNK = 320
N_CHUNKS = IDS_PER_W // CHUNK
NBUF = 2
assert N_CHUNKS % NBUF == 0


def _sc_gather(W, flat_ids):
    mesh = plsc.VectorSubcoreMesh(core_axis_name="c", subcore_axis_name="s")

    @functools.partial(
        pl.kernel,
        mesh=mesh,
        out_type=jax.ShapeDtypeStruct((NUM_IDS, DIM), W.dtype),
        scratch_types=[
            pltpu.VMEM((IDS_PER_W,), jnp.int32),
            pltpu.VMEM((NBUF, CHUNK, DIM), jnp.float32),
            pltpu.SemaphoreType.DMA((NBUF,)),
            pltpu.SemaphoreType.DMA((NBUF,)),
        ],
    )
    def gather_kernel(w_hbm, ids_hbm, out_hbm, idx_v, rows_v, gsem, osem):
        wid = lax.axis_index("s") * NC + lax.axis_index("c")
        base = wid * IDS_PER_W
        pltpu.sync_copy(ids_hbm.at[pl.ds(base, IDS_PER_W)], idx_v)

        def start_gather(k, b):
            pltpu.make_async_copy(
                w_hbm.at[idx_v.at[pl.ds(k * CHUNK, CHUNK)]],
                rows_v.at[b],
                gsem.at[b],
            ).start()

        def wait_gather(k, b):
            pltpu.make_async_copy(
                w_hbm.at[idx_v.at[pl.ds(k * CHUNK, CHUNK)]],
                rows_v.at[b],
                gsem.at[b],
            ).wait()

        def out_copy(k, b):
            return pltpu.make_async_copy(
                rows_v.at[b],
                out_hbm.at[pl.ds(base + k * CHUNK, CHUNK)],
                osem.at[b],
            )

        for b in range(NBUF):
            start_gather(b, b)

        @pl.loop(0, N_CHUNKS, step=NBUF)
        def _(c):
            for b in range(NBUF):
                k = c + b
                wait_gather(k, b)
                out_copy(k, b).start()

                @pl.when(k + NBUF < N_CHUNKS)
                def _():
                    out_copy(k, b).wait()
                    start_gather(k + NBUF, b)

        for b in range(NBUF):
            out_copy(N_CHUNKS - NBUF + b, b).wait()

    return gather_kernel(W, flat_ids)


def kernel(input_ids, attention_mask, W):
    flat_ids = input_ids.reshape(NUM_IDS)
    out = _sc_gather(W, flat_ids)
    token_embeddings = out.reshape(BATCH, SEQ, DIM)
    return (input_ids, token_embeddings, attention_mask)


# speedup vs baseline: 7.7970x; 1.0047x over previous
import functools

import jax
import jax.numpy as jnp
from jax import lax
from jax.experimental import pallas as pl
from jax.experimental.pallas import tpu as pltpu
from jax.experimental.pallas import tpu_sc as plsc

BATCH = 1024
SEQ = 200
DIM = 128

NUM_IDS = BATCH * SEQ
NC, NS = 2, 16
NW = NC * NS
IDS_PER_W = NUM_IDS // NW
CHUNK = 160
N_CHUNKS = IDS_PER_W // CHUNK
NBUF = 4
assert N_CHUNKS % NBUF == 0


def _sc_gather(W, flat_ids):
    mesh = plsc.VectorSubcoreMesh(core_axis_name="c", subcore_axis_name="s")

    @functools.partial(
        pl.kernel,
        mesh=mesh,
        out_type=jax.ShapeDtypeStruct((NUM_IDS, DIM), W.dtype),
        scratch_types=[
            pltpu.VMEM((IDS_PER_W,), jnp.int32),
            pltpu.VMEM((NBUF, CHUNK, DIM), jnp.float32),
            pltpu.SemaphoreType.DMA((NBUF,)),
            pltpu.SemaphoreType.DMA((NBUF,)),
        ],
    )
    def gather_kernel(w_hbm, ids_hbm, out_hbm, idx_v, rows_v, gsem, osem):
        wid = lax.axis_index("s") * NC + lax.axis_index("c")
        base = wid * IDS_PER_W
        pltpu.sync_copy(ids_hbm.at[pl.ds(base, IDS_PER_W)], idx_v)

        def start_gather(k, b):
            pltpu.make_async_copy(
                w_hbm.at[idx_v.at[pl.ds(k * CHUNK, CHUNK)]],
                rows_v.at[b],
                gsem.at[b],
            ).start()

        def wait_gather(k, b):
            pltpu.make_async_copy(
                w_hbm.at[idx_v.at[pl.ds(k * CHUNK, CHUNK)]],
                rows_v.at[b],
                gsem.at[b],
            ).wait()

        def out_copy(k, b):
            return pltpu.make_async_copy(
                rows_v.at[b],
                out_hbm.at[pl.ds(base + k * CHUNK, CHUNK)],
                osem.at[b],
            )

        for b in range(NBUF):
            start_gather(b, b)

        @pl.loop(0, N_CHUNKS, step=NBUF)
        def _(c):
            for b in range(NBUF):
                k = c + b
                wait_gather(k, b)
                out_copy(k, b).start()

                @pl.when(k + NBUF < N_CHUNKS)
                def _():
                    out_copy(k, b).wait()
                    start_gather(k + NBUF, b)

        for b in range(NBUF):
            out_copy(N_CHUNKS - NBUF + b, b).wait()

    return gather_kernel(W, flat_ids)


def kernel(input_ids, attention_mask, W):
    flat_ids = input_ids.reshape(NUM_IDS)
    out = _sc_gather(W, flat_ids)
    token_embeddings = out.reshape(BATCH, SEQ, DIM)
    return (input_ids, token_embeddings, attention_mask)
